# baseline (speedup 1.0000x reference)
import dataclasses
import functools

import jax
import jax.numpy as jnp
from jax import lax
from jax.experimental import pallas as pl
from jax.experimental.pallas import tpu as pltpu
from jax.experimental.pallas import tpu_sc as plsc

N = 10000
E = 320000
SIGMA = 0.3
V = 1.0

ROW_BLK = 2000

NC, NS, L = 2, 16, 16
NW = NC * NS
EB = 120
NSLOT = 3
NBLKG = -(-E // EB)
NBPW = -(-NBLKG // NW)
NPK = (NBPW + NSLOT) * NW
N_PAD = 10240
RPS = N_PAD // NS
ZR = 8


def _sc_compiler_params():
    cp = pltpu.CompilerParams()
    if "needs_layout_passes" in pltpu.CompilerParams.__dataclass_fields__:
        cp = dataclasses.replace(cp, needs_layout_passes=False)
    if "use_tc_tiling_on_sc" in pltpu.CompilerParams.__dataclass_fields__:
        cp = dataclasses.replace(cp, use_tc_tiling_on_sc=False)
    return cp


def _pack_edges(src, dst, w):
    pad = NPK * EB - E
    srcp = jnp.concatenate([src, jnp.zeros((pad,), jnp.int32)])
    dstp = jnp.concatenate([dst, jnp.zeros((pad,), jnp.int32)])
    wp = jnp.concatenate([w, jnp.zeros((pad,), jnp.float32)])
    pk = jnp.stack([srcp, dstp, lax.bitcast_convert_type(wp, jnp.int32)])
    return pk.reshape(3, NPK, EB).transpose(1, 0, 2)


def _spmm_sc(sup, pk):
    width = sup.shape[1]
    mesh = plsc.VectorSubcoreMesh(core_axis_name="c", subcore_axis_name="s")

    @functools.partial(
        pl.kernel,
        compiler_params=_sc_compiler_params(),
        out_type=jax.ShapeDtypeStruct((NC, N_PAD, width), jnp.float32),
        mesh=mesh,
        scratch_types=(
            [pltpu.VMEM((3, EB), jnp.int32) for _ in range(NSLOT)]
            + [pltpu.VMEM((EB, width), jnp.float32) for _ in range(NSLOT)]
            + [pltpu.VMEM((ZR, width), jnp.float32),
               pltpu.VMEM_SHARED((N_PAD, width), jnp.float32)]
            + [pltpu.SemaphoreType.DMA for _ in range(2 * NSLOT)]
        ),
    )
    def k(pk_h, sup_h, out_h, *scratch):
        pkv = scratch[:NSLOT]
        rows = scratch[NSLOT:2 * NSLOT]
        zrow = scratch[2 * NSLOT]
        acc = scratch[2 * NSLOT + 1]
        sem_i = scratch[2 * NSLOT + 2:2 * NSLOT + 2 + NSLOT]
        sem_g = scratch[2 * NSLOT + 2 + NSLOT:]
        cid = lax.axis_index("c")
        sid = lax.axis_index("s")
        wid = cid * NS + sid

        @pl.loop(0, ZR)
        def _(r):
            for c in range(width // L):
                zrow[r, pl.ds(c * L, L)] = jnp.zeros((L,), jnp.float32)

        for j in range(RPS // ZR):
            pltpu.sync_copy(zrow, acc.at[pl.ds(sid * RPS + j * ZR, ZR)])
        plsc.subcore_barrier()

        def g_of(b):
            return b * NW + wid

        def issue_idx(b, s):
            pltpu.async_copy(pk_h.at[g_of(b)], pkv[s], sem_i[s])

        def wait_idx(s):
            pltpu.make_async_copy(pk_h.at[0], pkv[s], sem_i[s]).wait()

        def issue_gather(s):
            pltpu.async_copy(sup_h.at[pkv[s].at[0]], rows[s], sem_g[s])

        def wait_gather(s):
            pltpu.make_async_copy(sup_h.at[pkv[s].at[0]], rows[s],
                                  sem_g[s]).wait()

        def scale(s):
            @plsc.parallel_loop(0, EB, unroll=4)
            def _(e):
                wi = plsc.load_gather(pkv[s].at[2],
                                      [jnp.full((L,), e, jnp.int32)])
                ws = plsc.bitcast(wi, jnp.float32)
                for c in range(width // L):
                    sl = pl.ds(c * L, L)
                    rows[s][e, sl] = rows[s][e, sl] * ws

        def scatter(s):
            pltpu.sync_copy(rows[s], acc.at[pkv[s].at[1]], add=True)

        def body(b, s):
            wait_gather(s)
            s2 = (s + 2) % NSLOT
            wait_idx(s2)
            issue_gather(s2)
            scale(s)
            scatter(s)
            issue_idx(b + NSLOT, s)

        for s in range(NSLOT):
            issue_idx(s, s)
        wait_idx(0)
        issue_gather(0)
        wait_idx(1)
        issue_gather(1)

        @pl.loop(0, NBPW, step=NSLOT)
        def _(b0):
            for s in range(NSLOT):
                body(b0 + s, s)

        wait_gather(0)
        wait_gather(1)
        wait_idx(2)

        plsc.subcore_barrier()
        pltpu.sync_copy(acc.at[pl.ds(sid * RPS, RPS)],
                        out_h.at[cid, pl.ds(sid * RPS, RPS)])

    parts = k(pk, sup)
    return parts[0, :N] + parts[1, :N]


def _spmm_chunks(sup_chunks, pk):
    return [_spmm_sc(c, pk) for c in sup_chunks]


def _encoder_body(x_ref, w1, b1, w2, b2, w3, b3, wz, bz,
                  tra1_ref, tra2_ref, tra3_ref, z_ref):
    x = x_ref[...]
    t1 = jax.nn.relu(jnp.dot(x, w1[...], preferred_element_type=jnp.float32) + b1[...])
    tra1_ref[...] = t1
    t2 = jax.nn.relu(jnp.dot(t1, w2[...], preferred_element_type=jnp.float32) + b2[...])
    tra2_ref[...] = t2
    t3 = jax.nn.relu(jnp.dot(t2, w3[...], preferred_element_type=jnp.float32) + b3[...])
    tra3_ref[...] = t3
    z_ref[...] = jnp.dot(t3, wz[...], preferred_element_type=jnp.float32) + bz[...]


def _encoder(x, enc1_W, enc1_b, enc2_W, enc2_b, enc3_W, enc3_b, zl_W, zl_b):
    D_IN, E1 = enc1_W.shape
    E2 = enc2_W.shape[1]
    E3 = enc3_W.shape[1]
    NZ = zl_W.shape[1]
    grid = (N // ROW_BLK,)
    full = lambda shape: pl.BlockSpec(shape, lambda i: (0,) * len(shape))
    row = lambda w: pl.BlockSpec((ROW_BLK, w), lambda i: (i, 0))
    return pl.pallas_call(
        _encoder_body,
        grid=grid,
        in_specs=[
            row(D_IN),
            full((D_IN, E1)), full((E1,)),
            full((E1, E2)), full((E2,)),
            full((E2, E3)), full((E3,)),
            full((E3, NZ)), full((NZ,)),
        ],
        out_specs=[row(E1), row(E2), row(E3), row(NZ)],
        out_shape=[
            jax.ShapeDtypeStruct((N, E1), jnp.float32),
            jax.ShapeDtypeStruct((N, E2), jnp.float32),
            jax.ShapeDtypeStruct((N, E3), jnp.float32),
            jax.ShapeDtypeStruct((N, NZ), jnp.float32),
        ],
    )(x, enc1_W, enc1_b, enc2_W, enc2_b, enc3_W, enc3_b, zl_W, zl_b)


def _row_call(body, in_shapes, out_widths):
    grid = (N // ROW_BLK,)
    in_specs = []
    for s in in_shapes:
        if s[0] == N:
            in_specs.append(pl.BlockSpec((ROW_BLK,) + s[1:],
                                         lambda i: (i,) + (0,) * (len(s) - 1)))
        else:
            in_specs.append(pl.BlockSpec(s, lambda i, _n=len(s): (0,) * _n))
    out_specs = [pl.BlockSpec((ROW_BLK, w), lambda i: (i, 0)) for w in out_widths]
    out_shape = [jax.ShapeDtypeStruct((N, w), jnp.float32) for w in out_widths]
    return pl.pallas_call(body, grid=grid, in_specs=in_specs,
                          out_specs=out_specs, out_shape=out_shape)


def _dot(a, b):
    return jnp.dot(a, b, preferred_element_type=jnp.float32)


def _mix_layer(m_chunks, gW, tra):
    nm = len(m_chunks)
    wout = gW.shape[1]

    def body(*refs):
        m_refs = refs[:nm]
        w_ref, tra_ref = refs[nm], refs[nm + 1]
        o_refs = refs[nm + 2:]
        acc = _dot(m_refs[0][...], w_ref[pl.ds(0, 128), :])
        for i in range(1, nm):
            acc += _dot(m_refs[i][...], w_ref[pl.ds(i * 128, 128), :])
        u = (1 - SIGMA) * jax.nn.relu(acc) + SIGMA * tra_ref[...]
        for j, o in enumerate(o_refs):
            o[...] = u[:, j * 128:(j + 1) * 128]

    in_shapes = [c.shape for c in m_chunks] + [gW.shape, tra.shape]
    return _row_call(body, in_shapes, [128] * (wout // 128))(
        *m_chunks, gW, tra)


def _layer3_project(m_chunks, g3W, tra3, g4W):
    nm = len(m_chunks)

    def body(*refs):
        m_refs = refs[:nm]
        w3, t3, w4, o = refs[nm], refs[nm + 1], refs[nm + 2], refs[nm + 3]
        acc = _dot(m_refs[0][...], w3[pl.ds(0, 128), :])
        for i in range(1, nm):
            acc += _dot(m_refs[i][...], w3[pl.ds(i * 128, 128), :])
        u = (1 - SIGMA) * jax.nn.relu(acc) + SIGMA * t3[...]
        o[...] = _dot(u, w4[...])

    in_shapes = [c.shape for c in m_chunks] + [g3W.shape, tra3.shape, g4W.shape]
    return _row_call(body, in_shapes, [g4W.shape[1]])(
        *m_chunks, g3W, tra3, g4W)[0]


def _layer5_project(h4, z, g5W):
    def body(h_ref, z_ref, w_ref, o_ref):
        u = (1 - SIGMA) * jax.nn.relu(h_ref[...]) + SIGMA * z_ref[...]
        o_ref[...] = _dot(u, w_ref[...])

    return _row_call(body, [h4.shape, z.shape, g5W.shape],
                     [g5W.shape[1]])(h4, z, g5W)[0]


def _tail(h4, h5, fc1_W, fc1_b, cluster):
    ncl = cluster.shape[0]

    def body(h4_ref, h5_ref, w_ref, b_ref, cl_ref, xb_ref, q_ref, pr_ref):
        h4v = h4_ref[...]
        h5v = h5_ref[...]
        r = jax.nn.relu(h4v)
        xb_ref[...] = jax.nn.relu(_dot(r, w_ref[...]) + b_ref[...])
        s = h5v - jnp.max(h5v, axis=1, keepdims=True)
        e = jnp.exp(s)
        pr_ref[...] = e / jnp.sum(e, axis=1, keepdims=True)
        cl = cl_ref[...]
        d = (jnp.sum(h4v * h4v, axis=1, keepdims=True)
             + jnp.sum(cl * cl, axis=1)[None, :]
             - 2.0 * _dot(h4v, cl.T))
        qv = 1.0 / (1.0 + d / V)
        q_ref[...] = qv / jnp.sum(qv, axis=1, keepdims=True)

    return _row_call(body,
                     [h4.shape, h5.shape, fc1_W.shape, fc1_b.shape,
                      cluster.shape],
                     [fc1_W.shape[1], ncl, h5.shape[1]])(
        h4, h5, fc1_W, fc1_b, cluster)


def kernel(x, edge_index, edge_weight, enc1_W, enc1_b, enc2_W, enc2_b, enc3_W, enc3_b,
           zl_W, zl_b, dec1_W, dec1_b, dec2_W, dec2_b, dec3_W, dec3_b, xbar_W, xbar_b,
           gnn1_W, gnn2_W, gnn3_W, gnn4_W, gnn5_W, fc1_W, fc1_b, cluster):
    tra1, tra2, tra3, z = _encoder(
        x, enc1_W, enc1_b, enc2_W, enc2_b, enc3_W, enc3_b, zl_W, zl_b)

    pk = _pack_edges(edge_index[0], edge_index[1], edge_weight)

    m1 = _spmm_chunks([x], pk)
    u2 = _mix_layer(m1, gnn1_W, tra1)
    m2 = _spmm_chunks(u2, pk)
    u3 = _mix_layer(m2, gnn2_W, tra2)
    m3 = _spmm_chunks(u3, pk)
    v4 = _layer3_project(m3, gnn3_W, tra3, gnn4_W)
    h4 = _spmm_chunks([v4], pk)[0]
    v5 = _layer5_project(h4, z, gnn5_W)
    h5 = _spmm_chunks([v5], pk)[0]

    x_bar, q, predict = _tail(h4, h5, fc1_W, fc1_b, cluster)

    return (x_bar, q, predict, z, h4, tra1, tra2, tra3)


# speedup vs baseline: 1.7449x; 1.7449x over previous
import dataclasses
import functools

import jax
import jax.numpy as jnp
from jax import lax
from jax.experimental import pallas as pl
from jax.experimental.pallas import tpu as pltpu
from jax.experimental.pallas import tpu_sc as plsc

N = 10000
E = 320000
SIGMA = 0.3
V = 1.0

ROW_BLK = 2000

NC, NS, L = 2, 16, 16
NW = NC * NS
EB = 120
NSLOT = 2
NBLKG = -(-E // EB)
NBPW = -(-NBLKG // NW)
NPK = (NBPW + NSLOT) * NW
N_PAD = 10240
RPS = N_PAD // NS
ZR = 8


def _sc_compiler_params():
    cp = pltpu.CompilerParams()
    if "needs_layout_passes" in pltpu.CompilerParams.__dataclass_fields__:
        cp = dataclasses.replace(cp, needs_layout_passes=False)
    if "use_tc_tiling_on_sc" in pltpu.CompilerParams.__dataclass_fields__:
        cp = dataclasses.replace(cp, use_tc_tiling_on_sc=False)
    return cp


def _pack_edges(src, dst, w):
    pad = NPK * EB - E
    srcp = jnp.concatenate([src, jnp.zeros((pad,), jnp.int32)])
    dstp = jnp.concatenate([dst, jnp.zeros((pad,), jnp.int32)])
    wp = jnp.concatenate([w, jnp.zeros((pad,), jnp.float32)])
    pk = jnp.stack([srcp, dstp, lax.bitcast_convert_type(wp, jnp.int32)])
    return pk.reshape(3, NPK, EB).transpose(1, 0, 2)


def _spmm_sc(sup, pk, packed16):
    if packed16:
        width = sup.shape[1] * 2
        gshape = (EB, width // 2)
        gdtype = jnp.int32
    else:
        width = sup.shape[1]
        gshape = (EB, width)
        gdtype = jnp.float32
    mesh = plsc.VectorSubcoreMesh(core_axis_name="c", subcore_axis_name="s")

    @functools.partial(
        pl.kernel,
        compiler_params=_sc_compiler_params(),
        out_type=jax.ShapeDtypeStruct((NC, N_PAD, width), jnp.float32),
        mesh=mesh,
        scratch_types=(
            [pltpu.VMEM((3, EB), jnp.int32) for _ in range(NSLOT)]
            + [pltpu.VMEM(gshape, gdtype) for _ in range(NSLOT)]
            + [pltpu.VMEM((EB, width), jnp.float32) for _ in range(NSLOT)]
            + [pltpu.VMEM((ZR, width), jnp.float32),
               pltpu.VMEM_SHARED((N_PAD, width), jnp.float32)]
            + [pltpu.SemaphoreType.DMA for _ in range(2 * NSLOT)]
        ),
    )
    def k(pk_h, sup_h, out_h, *scratch):
        pkv = scratch[:NSLOT]
        rows = scratch[NSLOT:2 * NSLOT]
        stg = scratch[2 * NSLOT:3 * NSLOT]
        zrow = scratch[3 * NSLOT]
        acc = scratch[3 * NSLOT + 1]
        sem_i = scratch[3 * NSLOT + 2:3 * NSLOT + 2 + NSLOT]
        sem_g = scratch[3 * NSLOT + 2 + NSLOT:]
        cid = lax.axis_index("c")
        sid = lax.axis_index("s")
        wid = cid * NS + sid

        @pl.loop(0, ZR)
        def _(r):
            for c in range(width // L):
                zrow[r, pl.ds(c * L, L)] = jnp.zeros((L,), jnp.float32)

        for j in range(RPS // ZR):
            pltpu.sync_copy(zrow, acc.at[pl.ds(sid * RPS + j * ZR, ZR)])
        plsc.subcore_barrier()

        def g_of(b):
            return b * NW + wid

        def issue_idx(b, s):
            pltpu.async_copy(pk_h.at[g_of(b)], pkv[s], sem_i[s])

        def wait_idx(s):
            pltpu.make_async_copy(pk_h.at[0], pkv[s], sem_i[s]).wait()

        def issue_gather(s):
            pltpu.async_copy(sup_h.at[pkv[s].at[0]], rows[s], sem_g[s])

        def wait_gather(s):
            pltpu.make_async_copy(sup_h.at[pkv[s].at[0]], rows[s],
                                  sem_g[s]).wait()

        def scale(s):
            @plsc.parallel_loop(0, EB, unroll=4)
            def _(e):
                wi = plsc.load_gather(pkv[s].at[2],
                                      [jnp.full((L,), e, jnp.int32)])
                ws = plsc.bitcast(wi, jnp.float32)
                if packed16:
                    mask = jnp.full((L,), -65536, jnp.int32)
                    for c in range(width // (2 * L)):
                        w16 = rows[s][e, pl.ds(c * L, L)]
                        lo = plsc.bitcast(w16 << 16, jnp.float32)
                        hi = plsc.bitcast(w16 & mask, jnp.float32)
                        stg[s][e, pl.ds(2 * c * L, L)] = lo * ws
                        stg[s][e, pl.ds((2 * c + 1) * L, L)] = hi * ws
                else:
                    for c in range(width // L):
                        sl = pl.ds(c * L, L)
                        stg[s][e, sl] = rows[s][e, sl] * ws

        def scatter(s):
            pltpu.sync_copy(stg[s], acc.at[pkv[s].at[1]], add=True)

        def body(b, s):
            wait_gather(s)
            wait_idx(1 - s)
            issue_gather(1 - s)
            scale(s)
            scatter(s)
            issue_idx(b + 2, s)

        issue_idx(0, 0)
        issue_idx(1, 1)
        wait_idx(0)
        issue_gather(0)

        @pl.loop(0, NBPW, step=2)
        def _(b0):
            body(b0, 0)
            body(b0 + 1, 1)

        wait_gather(0)
        wait_idx(1)

        plsc.subcore_barrier()
        pltpu.sync_copy(acc.at[pl.ds(sid * RPS, RPS)],
                        out_h.at[cid, pl.ds(sid * RPS, RPS)])

    parts = k(pk, sup)
    return parts[0, :N] + parts[1, :N]


def _spmm_chunks(sup_chunks, pk, packed16):
    return [_spmm_sc(c, pk, packed16) for c in sup_chunks]


def _pack_cols(u):
    W = u.shape[1]
    lo = jnp.concatenate([u[:, c * 32:c * 32 + 16] for c in range(W // 32)],
                         axis=1)
    hi = jnp.concatenate([u[:, c * 32 + 16:c * 32 + 32] for c in range(W // 32)],
                         axis=1)
    lo_i = lax.bitcast_convert_type(lo.astype(jnp.bfloat16),
                                    jnp.uint16).astype(jnp.int32)
    hi_i = lax.bitcast_convert_type(hi.astype(jnp.bfloat16),
                                    jnp.uint16).astype(jnp.int32)
    return lo_i | (hi_i << 16)


def _encoder_body(x_ref, w1, b1, w2, b2, w3, b3, wz, bz,
                  tra1_ref, tra2_ref, tra3_ref, z_ref, xp_ref):
    x = x_ref[...]
    t1 = jax.nn.relu(jnp.dot(x, w1[...], preferred_element_type=jnp.float32) + b1[...])
    tra1_ref[...] = t1
    t2 = jax.nn.relu(jnp.dot(t1, w2[...], preferred_element_type=jnp.float32) + b2[...])
    tra2_ref[...] = t2
    t3 = jax.nn.relu(jnp.dot(t2, w3[...], preferred_element_type=jnp.float32) + b3[...])
    tra3_ref[...] = t3
    z_ref[...] = jnp.dot(t3, wz[...], preferred_element_type=jnp.float32) + bz[...]
    xp_ref[...] = _pack_cols(x)


def _encoder(x, enc1_W, enc1_b, enc2_W, enc2_b, enc3_W, enc3_b, zl_W, zl_b):
    D_IN, E1 = enc1_W.shape
    E2 = enc2_W.shape[1]
    E3 = enc3_W.shape[1]
    NZ = zl_W.shape[1]
    grid = (N // ROW_BLK,)
    full = lambda shape: pl.BlockSpec(shape, lambda i: (0,) * len(shape))
    row = lambda w: pl.BlockSpec((ROW_BLK, w), lambda i: (i, 0))
    return pl.pallas_call(
        _encoder_body,
        grid=grid,
        in_specs=[
            row(D_IN),
            full((D_IN, E1)), full((E1,)),
            full((E1, E2)), full((E2,)),
            full((E2, E3)), full((E3,)),
            full((E3, NZ)), full((NZ,)),
        ],
        out_specs=[row(E1), row(E2), row(E3), row(NZ), row(D_IN // 2)],
        out_shape=[
            jax.ShapeDtypeStruct((N, E1), jnp.float32),
            jax.ShapeDtypeStruct((N, E2), jnp.float32),
            jax.ShapeDtypeStruct((N, E3), jnp.float32),
            jax.ShapeDtypeStruct((N, NZ), jnp.float32),
            jax.ShapeDtypeStruct((N, D_IN // 2), jnp.int32),
        ],
    )(x, enc1_W, enc1_b, enc2_W, enc2_b, enc3_W, enc3_b, zl_W, zl_b)


def _row_call(body, in_shapes, out_widths):
    grid = (N // ROW_BLK,)
    in_specs = []
    for s in in_shapes:
        if s[0] == N:
            in_specs.append(pl.BlockSpec((ROW_BLK,) + s[1:],
                                         lambda i: (i,) + (0,) * (len(s) - 1)))
        else:
            in_specs.append(pl.BlockSpec(s, lambda i, _n=len(s): (0,) * _n))
    out_specs = [pl.BlockSpec((ROW_BLK, w), lambda i: (i, 0))
                 for w, _ in out_widths]
    out_shape = [jax.ShapeDtypeStruct((N, w), dt) for w, dt in out_widths]
    return pl.pallas_call(body, grid=grid, in_specs=in_specs,
                          out_specs=out_specs, out_shape=out_shape)


def _dot(a, b):
    return jnp.dot(a, b, preferred_element_type=jnp.float32)


def _mix_layer(m_chunks, gW, tra):
    nm = len(m_chunks)
    wout = gW.shape[1]

    def body(*refs):
        m_refs = refs[:nm]
        w_ref, tra_ref = refs[nm], refs[nm + 1]
        o_refs = refs[nm + 2:]
        acc = _dot(m_refs[0][...], w_ref[pl.ds(0, 128), :])
        for i in range(1, nm):
            acc += _dot(m_refs[i][...], w_ref[pl.ds(i * 128, 128), :])
        u = (1 - SIGMA) * jax.nn.relu(acc) + SIGMA * tra_ref[...]
        for j, o in enumerate(o_refs):
            o[...] = _pack_cols(u[:, j * 128:(j + 1) * 128])

    in_shapes = [c.shape for c in m_chunks] + [gW.shape, tra.shape]
    return _row_call(body, in_shapes, [(64, jnp.int32)] * (wout // 128))(
        *m_chunks, gW, tra)


def _layer3_project(m_chunks, g3W, tra3, g4W):
    nm = len(m_chunks)

    def body(*refs):
        m_refs = refs[:nm]
        w3, t3, w4, o = refs[nm], refs[nm + 1], refs[nm + 2], refs[nm + 3]
        acc = _dot(m_refs[0][...], w3[pl.ds(0, 128), :])
        for i in range(1, nm):
            acc += _dot(m_refs[i][...], w3[pl.ds(i * 128, 128), :])
        u = (1 - SIGMA) * jax.nn.relu(acc) + SIGMA * t3[...]
        o[...] = _dot(u, w4[...])

    in_shapes = [c.shape for c in m_chunks] + [g3W.shape, tra3.shape, g4W.shape]
    return _row_call(body, in_shapes, [(g4W.shape[1], jnp.float32)])(
        *m_chunks, g3W, tra3, g4W)[0]


def _layer5_project(h4, z, g5W):
    def body(h_ref, z_ref, w_ref, o_ref):
        u = (1 - SIGMA) * jax.nn.relu(h_ref[...]) + SIGMA * z_ref[...]
        o_ref[...] = _dot(u, w_ref[...])

    return _row_call(body, [h4.shape, z.shape, g5W.shape],
                     [(g5W.shape[1], jnp.float32)])(h4, z, g5W)[0]


def _tail(h4, h5, fc1_W, fc1_b, cluster):
    ncl = cluster.shape[0]

    def body(h4_ref, h5_ref, w_ref, b_ref, cl_ref, xb_ref, q_ref, pr_ref):
        h4v = h4_ref[...]
        h5v = h5_ref[...]
        r = jax.nn.relu(h4v)
        xb_ref[...] = jax.nn.relu(_dot(r, w_ref[...]) + b_ref[...])
        s = h5v - jnp.max(h5v, axis=1, keepdims=True)
        e = jnp.exp(s)
        pr_ref[...] = e / jnp.sum(e, axis=1, keepdims=True)
        cl = cl_ref[...]
        d = (jnp.sum(h4v * h4v, axis=1, keepdims=True)
             + jnp.sum(cl * cl, axis=1)[None, :]
             - 2.0 * _dot(h4v, cl.T))
        qv = 1.0 / (1.0 + d / V)
        q_ref[...] = qv / jnp.sum(qv, axis=1, keepdims=True)

    return _row_call(body,
                     [h4.shape, h5.shape, fc1_W.shape, fc1_b.shape,
                      cluster.shape],
                     [(fc1_W.shape[1], jnp.float32), (ncl, jnp.float32),
                      (h5.shape[1], jnp.float32)])(
        h4, h5, fc1_W, fc1_b, cluster)


def kernel(x, edge_index, edge_weight, enc1_W, enc1_b, enc2_W, enc2_b, enc3_W, enc3_b,
           zl_W, zl_b, dec1_W, dec1_b, dec2_W, dec2_b, dec3_W, dec3_b, xbar_W, xbar_b,
           gnn1_W, gnn2_W, gnn3_W, gnn4_W, gnn5_W, fc1_W, fc1_b, cluster):
    tra1, tra2, tra3, z, xp = _encoder(
        x, enc1_W, enc1_b, enc2_W, enc2_b, enc3_W, enc3_b, zl_W, zl_b)

    pk = _pack_edges(edge_index[0], edge_index[1], edge_weight)

    m1 = _spmm_chunks([xp], pk, packed16=True)
    u2 = _mix_layer(m1, gnn1_W, tra1)
    m2 = _spmm_chunks(u2, pk, packed16=True)
    u3 = _mix_layer(m2, gnn2_W, tra2)
    m3 = _spmm_chunks(u3, pk, packed16=True)
    v4 = _layer3_project(m3, gnn3_W, tra3, gnn4_W)
    h4 = _spmm_chunks([v4], pk, packed16=False)[0]
    v5 = _layer5_project(h4, z, gnn5_W)
    h5 = _spmm_chunks([v5], pk, packed16=False)[0]

    x_bar, q, predict = _tail(h4, h5, fc1_W, fc1_b, cluster)

    return (x_bar, q, predict, z, h4, tra1, tra2, tra3)


# speedup vs baseline: 1.7866x; 1.0239x over previous
import dataclasses
import functools

import jax
import jax.numpy as jnp
from jax import lax
from jax.experimental import pallas as pl
from jax.experimental.pallas import tpu as pltpu
from jax.experimental.pallas import tpu_sc as plsc

N = 10000
E = 320000
SIGMA = 0.3
V = 1.0

ROW_BLK = 2000

NC, NS, L = 2, 16, 16
NW = NC * NS
EB = 120
NSLOT = 2
NBLKG = -(-E // EB)
NBPW = -(-NBLKG // NW)
NPK = (NBPW + NSLOT) * NW
N_PAD = 10240
RPS = N_PAD // NS
ZR = 8


def _sc_compiler_params():
    cp = pltpu.CompilerParams()
    if "needs_layout_passes" in pltpu.CompilerParams.__dataclass_fields__:
        cp = dataclasses.replace(cp, needs_layout_passes=False)
    if "use_tc_tiling_on_sc" in pltpu.CompilerParams.__dataclass_fields__:
        cp = dataclasses.replace(cp, use_tc_tiling_on_sc=False)
    return cp


def _pack_edges(src, dst, w):
    pad = NPK * EB - E
    srcp = jnp.concatenate([src, jnp.zeros((pad,), jnp.int32)])
    dstp = jnp.concatenate([dst, jnp.zeros((pad,), jnp.int32)])
    wp = jnp.concatenate([w, jnp.zeros((pad,), jnp.float32)])
    pk = jnp.stack([srcp, dstp, lax.bitcast_convert_type(wp, jnp.int32)])
    return pk.reshape(3, NPK, EB).transpose(1, 0, 2)


def _spmm_sc(sup, pk, packed16):
    if packed16:
        width = sup.shape[1] * 2
        gshape = (EB, width // 2)
        gdtype = jnp.int32
    else:
        width = sup.shape[1]
        gshape = (EB, width)
        gdtype = jnp.float32
    mesh = plsc.VectorSubcoreMesh(core_axis_name="c", subcore_axis_name="s")

    @functools.partial(
        pl.kernel,
        compiler_params=_sc_compiler_params(),
        out_type=jax.ShapeDtypeStruct((NC, N_PAD, width), jnp.float32),
        mesh=mesh,
        scratch_types=(
            [pltpu.VMEM((3, EB), jnp.int32) for _ in range(NSLOT)]
            + [pltpu.VMEM(gshape, gdtype) for _ in range(NSLOT)]
            + [pltpu.VMEM((EB, width), jnp.float32) for _ in range(NSLOT)]
            + [pltpu.VMEM((ZR, width), jnp.float32),
               pltpu.VMEM_SHARED((N_PAD, width), jnp.float32)]
            + [pltpu.SemaphoreType.DMA for _ in range(2 * NSLOT)]
        ),
    )
    def k(pk_h, sup_h, out_h, *scratch):
        pkv = scratch[:NSLOT]
        rows = scratch[NSLOT:2 * NSLOT]
        stg = scratch[2 * NSLOT:3 * NSLOT]
        zrow = scratch[3 * NSLOT]
        acc = scratch[3 * NSLOT + 1]
        sem_i = scratch[3 * NSLOT + 2:3 * NSLOT + 2 + NSLOT]
        sem_g = scratch[3 * NSLOT + 2 + NSLOT:]
        cid = lax.axis_index("c")
        sid = lax.axis_index("s")
        wid = cid * NS + sid

        @pl.loop(0, ZR)
        def _(r):
            for c in range(width // L):
                zrow[r, pl.ds(c * L, L)] = jnp.zeros((L,), jnp.float32)

        for j in range(RPS // ZR):
            pltpu.sync_copy(zrow, acc.at[pl.ds(sid * RPS + j * ZR, ZR)])
        plsc.subcore_barrier()

        def g_of(b):
            return b * NW + wid

        def issue_idx(b, s):
            pltpu.async_copy(pk_h.at[g_of(b)], pkv[s], sem_i[s])

        def wait_idx(s):
            pltpu.make_async_copy(pk_h.at[0], pkv[s], sem_i[s]).wait()

        def issue_gather(s):
            pltpu.async_copy(sup_h.at[pkv[s].at[0]], rows[s], sem_g[s])

        def wait_gather(s):
            pltpu.make_async_copy(sup_h.at[pkv[s].at[0]], rows[s],
                                  sem_g[s]).wait()

        def scale(s):
            @plsc.parallel_loop(0, EB, unroll=4)
            def _(e):
                wi = plsc.load_gather(pkv[s].at[2],
                                      [jnp.full((L,), e, jnp.int32)])
                ws = plsc.bitcast(wi, jnp.float32)
                if packed16:
                    mask = jnp.full((L,), -65536, jnp.int32)
                    for c in range(width // (2 * L)):
                        w16 = rows[s][e, pl.ds(c * L, L)]
                        lo = plsc.bitcast(w16 << 16, jnp.float32)
                        hi = plsc.bitcast(w16 & mask, jnp.float32)
                        stg[s][e, pl.ds(2 * c * L, L)] = lo * ws
                        stg[s][e, pl.ds((2 * c + 1) * L, L)] = hi * ws
                else:
                    for c in range(width // L):
                        sl = pl.ds(c * L, L)
                        stg[s][e, sl] = rows[s][e, sl] * ws

        def scatter(s):
            pltpu.sync_copy(stg[s], acc.at[pkv[s].at[1]], add=True)

        def body(b, s):
            wait_gather(s)
            wait_idx(1 - s)
            issue_gather(1 - s)
            scale(s)
            scatter(s)
            issue_idx(b + 2, s)

        issue_idx(0, 0)
        issue_idx(1, 1)
        wait_idx(0)
        issue_gather(0)

        @pl.loop(0, NBPW, step=2)
        def _(b0):
            body(b0, 0)
            body(b0 + 1, 1)

        wait_gather(0)
        wait_idx(1)

        plsc.subcore_barrier()
        pltpu.sync_copy(acc.at[pl.ds(sid * RPS, RPS)],
                        out_h.at[cid, pl.ds(sid * RPS, RPS)])

    parts = k(pk, sup)
    return parts[0, :N] + parts[1, :N]


def _spmm_chunks(sup_chunks, pk, packed16):
    return [_spmm_sc(c, pk, packed16) for c in sup_chunks]


def _pack_cols(u):
    W = u.shape[1]
    lo = jnp.concatenate([u[:, c * 32:c * 32 + 16] for c in range(W // 32)],
                         axis=1)
    hi = jnp.concatenate([u[:, c * 32 + 16:c * 32 + 32] for c in range(W // 32)],
                         axis=1)
    lo_i = lax.bitcast_convert_type(lo.astype(jnp.bfloat16),
                                    jnp.uint16).astype(jnp.int32)
    hi_i = lax.bitcast_convert_type(hi.astype(jnp.bfloat16),
                                    jnp.uint16).astype(jnp.int32)
    return lo_i | (hi_i << 16)


def _encoder_body(x_ref, w1, b1, w2, b2, w3, b3, wz, bz,
                  tra1_ref, tra2_ref, tra3_ref, z_ref):
    x = x_ref[...]
    t1 = jax.nn.relu(jnp.dot(x, w1[...], preferred_element_type=jnp.float32) + b1[...])
    tra1_ref[...] = t1
    t2 = jax.nn.relu(jnp.dot(t1, w2[...], preferred_element_type=jnp.float32) + b2[...])
    tra2_ref[...] = t2
    t3 = jax.nn.relu(jnp.dot(t2, w3[...], preferred_element_type=jnp.float32) + b3[...])
    tra3_ref[...] = t3
    z_ref[...] = jnp.dot(t3, wz[...], preferred_element_type=jnp.float32) + bz[...]


def _encoder(x, enc1_W, enc1_b, enc2_W, enc2_b, enc3_W, enc3_b, zl_W, zl_b):
    D_IN, E1 = enc1_W.shape
    E2 = enc2_W.shape[1]
    E3 = enc3_W.shape[1]
    NZ = zl_W.shape[1]
    grid = (N // ROW_BLK,)
    full = lambda shape: pl.BlockSpec(shape, lambda i: (0,) * len(shape))
    row = lambda w: pl.BlockSpec((ROW_BLK, w), lambda i: (i, 0))
    return pl.pallas_call(
        _encoder_body,
        grid=grid,
        in_specs=[
            row(D_IN),
            full((D_IN, E1)), full((E1,)),
            full((E1, E2)), full((E2,)),
            full((E2, E3)), full((E3,)),
            full((E3, NZ)), full((NZ,)),
        ],
        out_specs=[row(E1), row(E2), row(E3), row(NZ)],
        out_shape=[
            jax.ShapeDtypeStruct((N, E1), jnp.float32),
            jax.ShapeDtypeStruct((N, E2), jnp.float32),
            jax.ShapeDtypeStruct((N, E3), jnp.float32),
            jax.ShapeDtypeStruct((N, NZ), jnp.float32),
        ],
    )(x, enc1_W, enc1_b, enc2_W, enc2_b, enc3_W, enc3_b, zl_W, zl_b)


def _row_call(body, in_shapes, out_widths):
    grid = (N // ROW_BLK,)
    in_specs = []
    for s in in_shapes:
        if s[0] == N:
            in_specs.append(pl.BlockSpec((ROW_BLK,) + s[1:],
                                         lambda i: (i,) + (0,) * (len(s) - 1)))
        else:
            in_specs.append(pl.BlockSpec(s, lambda i, _n=len(s): (0,) * _n))
    out_specs = [pl.BlockSpec((ROW_BLK, w), lambda i: (i, 0))
                 for w, _ in out_widths]
    out_shape = [jax.ShapeDtypeStruct((N, w), dt) for w, dt in out_widths]
    return pl.pallas_call(body, grid=grid, in_specs=in_specs,
                          out_specs=out_specs, out_shape=out_shape)


def _dot(a, b):
    return jnp.dot(a, b, preferred_element_type=jnp.float32)


def _mix_layer(m_chunks, gW, tra):
    nm = len(m_chunks)
    wout = gW.shape[1]

    def body(*refs):
        m_refs = refs[:nm]
        w_ref, tra_ref = refs[nm], refs[nm + 1]
        o_refs = refs[nm + 2:]
        acc = _dot(m_refs[0][...], w_ref[pl.ds(0, 128), :])
        for i in range(1, nm):
            acc += _dot(m_refs[i][...], w_ref[pl.ds(i * 128, 128), :])
        u = (1 - SIGMA) * jax.nn.relu(acc) + SIGMA * tra_ref[...]
        for j, o in enumerate(o_refs):
            o[...] = _pack_cols(u[:, j * 128:(j + 1) * 128])

    in_shapes = [c.shape for c in m_chunks] + [gW.shape, tra.shape]
    return _row_call(body, in_shapes, [(64, jnp.int32)] * (wout // 128))(
        *m_chunks, gW, tra)


def _layer3_project(m_chunks, g3W, tra3, g4W):
    nm = len(m_chunks)

    def body(*refs):
        m_refs = refs[:nm]
        w3, t3, w4, o = refs[nm], refs[nm + 1], refs[nm + 2], refs[nm + 3]
        acc = _dot(m_refs[0][...], w3[pl.ds(0, 128), :])
        for i in range(1, nm):
            acc += _dot(m_refs[i][...], w3[pl.ds(i * 128, 128), :])
        u = (1 - SIGMA) * jax.nn.relu(acc) + SIGMA * t3[...]
        o[...] = _pack_cols(_dot(u, w4[...]))

    in_shapes = [c.shape for c in m_chunks] + [g3W.shape, tra3.shape, g4W.shape]
    return _row_call(body, in_shapes, [(g4W.shape[1] // 2, jnp.int32)])(
        *m_chunks, g3W, tra3, g4W)[0]


def _layer5_project(h4, z, g5W):
    def body(h_ref, z_ref, w_ref, o_ref):
        u = (1 - SIGMA) * jax.nn.relu(h_ref[...]) + SIGMA * z_ref[...]
        o_ref[...] = _dot(u, w_ref[...])

    return _row_call(body, [h4.shape, z.shape, g5W.shape],
                     [(g5W.shape[1], jnp.float32)])(h4, z, g5W)[0]


def _tail(h4, h5, fc1_W, fc1_b, cluster):
    ncl = cluster.shape[0]

    def body(h4_ref, h5_ref, w_ref, b_ref, cl_ref, xb_ref, q_ref, pr_ref):
        h4v = h4_ref[...]
        h5v = h5_ref[...]
        r = jax.nn.relu(h4v)
        xb_ref[...] = jax.nn.relu(_dot(r, w_ref[...]) + b_ref[...])
        s = h5v - jnp.max(h5v, axis=1, keepdims=True)
        e = jnp.exp(s)
        pr_ref[...] = e / jnp.sum(e, axis=1, keepdims=True)
        cl = cl_ref[...]
        d = (jnp.sum(h4v * h4v, axis=1, keepdims=True)
             + jnp.sum(cl * cl, axis=1)[None, :]
             - 2.0 * _dot(h4v, cl.T))
        qv = 1.0 / (1.0 + d / V)
        q_ref[...] = qv / jnp.sum(qv, axis=1, keepdims=True)

    return _row_call(body,
                     [h4.shape, h5.shape, fc1_W.shape, fc1_b.shape,
                      cluster.shape],
                     [(fc1_W.shape[1], jnp.float32), (ncl, jnp.float32),
                      (h5.shape[1], jnp.float32)])(
        h4, h5, fc1_W, fc1_b, cluster)


def kernel(x, edge_index, edge_weight, enc1_W, enc1_b, enc2_W, enc2_b, enc3_W, enc3_b,
           zl_W, zl_b, dec1_W, dec1_b, dec2_W, dec2_b, dec3_W, dec3_b, xbar_W, xbar_b,
           gnn1_W, gnn2_W, gnn3_W, gnn4_W, gnn5_W, fc1_W, fc1_b, cluster):
    tra1, tra2, tra3, z = _encoder(
        x, enc1_W, enc1_b, enc2_W, enc2_b, enc3_W, enc3_b, zl_W, zl_b)

    pk = _pack_edges(edge_index[0], edge_index[1], edge_weight)

    def _pack_x_body(x_ref, o_ref):
        o_ref[...] = _pack_cols(x_ref[...])

    xp = _row_call(_pack_x_body, [x.shape],
                   [(x.shape[1] // 2, jnp.int32)])(x)[0]

    m1 = _spmm_chunks([xp], pk, packed16=True)
    u2 = _mix_layer(m1, gnn1_W, tra1)
    m2 = _spmm_chunks(u2, pk, packed16=True)
    u3 = _mix_layer(m2, gnn2_W, tra2)
    m3 = _spmm_chunks(u3, pk, packed16=True)
    v4 = _layer3_project(m3, gnn3_W, tra3, gnn4_W)
    h4 = _spmm_chunks([v4], pk, packed16=True)[0]
    v5 = _layer5_project(h4, z, gnn5_W)
    h5 = _spmm_chunks([v5], pk, packed16=False)[0]

    x_bar, q, predict = _tail(h4, h5, fc1_W, fc1_b, cluster)

    return (x_bar, q, predict, z, h4, tra1, tra2, tra3)
